# baseline (speedup 1.0000x reference)
import functools

import jax
import jax.numpy as jnp
from jax import lax
from jax.experimental import pallas as pl
from jax.experimental.pallas import tpu as pltpu
from jax.experimental.pallas import tpu_sc as plsc

BATCH = 16384
D = 32
LANES = 16
NBAND = 4
NSUB = 8
CHUNK = 16

_info = plsc.get_sparse_core_info()
NC = _info.num_cores
NS = _info.num_subcores
NW = NC * NS
BPW = BATCH // NW
NCHUNK = BPW // CHUNK

_mesh = plsc.VectorSubcoreMesh(core_axis_name="c", subcore_axis_name="s")


@functools.partial(
    pl.kernel,
    mesh=_mesh,
    out_type=jax.ShapeDtypeStruct((BATCH,), jnp.float32),
    scratch_types=[
        pltpu.VMEM((BPW,), jnp.int32),
        pltpu.VMEM((BPW,), jnp.int32),
        pltpu.VMEM((NBAND, NSUB, CHUNK * LANES), jnp.float32),
        pltpu.VMEM((NBAND, NSUB, CHUNK * LANES), jnp.float32),
        pltpu.VMEM((NBAND, NSUB, CHUNK * LANES), jnp.float32),
        pltpu.VMEM((NBAND, NSUB, CHUNK * LANES), jnp.float32),
        pltpu.VMEM((3 * LANES,), jnp.float32),
        pltpu.VMEM((BPW,), jnp.float32),
        pltpu.SemaphoreType.DMA,
        pltpu.SemaphoreType.DMA,
        pltpu.SemaphoreType.DMA,
        pltpu.SemaphoreType.DMA,
    ],
    compiler_params=pltpu.CompilerParams(needs_layout_passes=False),
)
def _sc_fused(uid_hbm, mid_hbm, utab_hbm, mtab_hbm, wb_hbm, out_hbm,
              uidx_v, midx_v, ua_v, ub_v, ma_v, mb_v, wb_v, out_v,
              usem, msem, usem2, msem2):
    wid = lax.axis_index("s") * NC + lax.axis_index("c")
    base = wid * BPW

    pltpu.sync_copy(wb_hbm, wb_v)
    pltpu.sync_copy(uid_hbm.at[pl.ds(base, BPW)], uidx_v)
    pltpu.sync_copy(mid_hbm.at[pl.ds(base, BPW)], midx_v)

    wv0 = wb_v[pl.ds(0, LANES)]
    wv1 = wb_v[pl.ds(LANES, LANES)]
    bias_vec = wb_v[pl.ds(2 * LANES, LANES)]
    lane_base = lax.iota(jnp.int32, LANES) * LANES

    def fire(c, urows_v, mrows_v):
        coff = pl.multiple_of(c * CHUNK, LANES)
        for g in range(CHUNK // LANES):
            uvec = uidx_v[pl.ds(coff + g * LANES, LANES)]
            mvec = midx_v[pl.ds(coff + g * LANES, LANES)]
            for j in range(NBAND):
                for l in range(LANES):
                    i = g * LANES + l
                    pltpu.async_copy(
                        utab_hbm.at[j, :, pl.ds((uvec[l] // LANES) * LANES,
                                                LANES)],
                        urows_v.at[j, :, pl.ds(i * LANES, LANES)],
                        usem if l % 2 == 0 else usem2)
                    pltpu.async_copy(
                        mtab_hbm.at[j, :, pl.ds((mvec[l] // LANES) * LANES,
                                                LANES)],
                        mrows_v.at[j, :, pl.ds(i * LANES, LANES)],
                        msem if l % 2 == 0 else msem2)

    def drain_compute(c, urows_v, mrows_v):
        half = CHUNK * LANES // 2
        pltpu.make_async_copy(
            utab_hbm.at[:, :, pl.ds(0, half)],
            urows_v.at[:, :, pl.ds(0, half)], usem).wait()
        pltpu.make_async_copy(
            utab_hbm.at[:, :, pl.ds(0, half)],
            urows_v.at[:, :, pl.ds(0, half)], usem2).wait()
        pltpu.make_async_copy(
            mtab_hbm.at[:, :, pl.ds(0, half)],
            mrows_v.at[:, :, pl.ds(0, half)], msem).wait()
        pltpu.make_async_copy(
            mtab_hbm.at[:, :, pl.ds(0, half)],
            mrows_v.at[:, :, pl.ds(0, half)], msem2).wait()
        coff = pl.multiple_of(c * CHUNK, LANES)
        for g in range(CHUNK // LANES):
            uvec = uidx_v[pl.ds(coff + g * LANES, LANES)]
            mvec = midx_v[pl.ds(coff + g * LANES, LANES)]
            uoffs = lane_base + g * (LANES * LANES) + (uvec & (LANES - 1))
            moffs = lane_base + g * (LANES * LANES) + (mvec & (LANES - 1))
            acc = bias_vec
            for j in range(NBAND):
                for s in range(NSUB):
                    d = j * NSUB + s
                    jv = jnp.full((LANES,), j, jnp.int32)
                    sv = jnp.full((LANES,), s, jnp.int32)
                    uv = plsc.load_gather(urows_v, [jv, sv, uoffs])
                    mv = plsc.load_gather(mrows_v, [jv, sv, moffs])
                    wsrc = wv0 if d < LANES else wv1
                    wd = jnp.broadcast_to(wsrc[d % LANES], (LANES,))
                    acc = acc + uv * mv * wd
            out_v[pl.ds(coff + g * LANES, LANES)] = acc

    fire(0, ua_v, ma_v)

    def pair(c2, _):
        c = c2 * 2
        fire(c + 1, ub_v, mb_v)
        drain_compute(c, ua_v, ma_v)
        @pl.when(c + 2 < NCHUNK)
        def _():
            fire(c + 2, ua_v, ma_v)
        drain_compute(c + 1, ub_v, mb_v)
        return 0

    lax.fori_loop(0, NCHUNK // 2, pair, 0)

    pltpu.sync_copy(out_v, out_hbm.at[pl.ds(base, BPW)])


def kernel(user_id, movie_id, user_table, movie_table, dense_W, dense_b):
    n_users = user_table.shape[0]
    n_movies = movie_table.shape[0]
    u3 = user_table.reshape(n_users, NBAND, NSUB).transpose(1, 2, 0)
    m3 = movie_table.reshape(n_movies, NBAND, NSUB).transpose(1, 2, 0)
    wb = jnp.concatenate([
        dense_W.reshape(-1),
        jnp.broadcast_to(dense_b.reshape(-1), (LANES,)),
    ])
    out = _sc_fused(user_id, movie_id, u3, m3, wb)
    return out.reshape(BATCH, 1)


# speedup vs baseline: 1.1552x; 1.1552x over previous
import functools

import jax
import jax.numpy as jnp
from jax import lax
from jax.experimental import pallas as pl
from jax.experimental.pallas import tpu as pltpu
from jax.experimental.pallas import tpu_sc as plsc

BATCH = 16384
D = 32
LANES = 16
NBAND = 4
NSUB = 8
CHUNK = 16
MCHUNK = 128

_info = plsc.get_sparse_core_info()
NC = _info.num_cores
NS = _info.num_subcores
NW = NC * NS
BPW = BATCH // NW
NCHUNK = BPW // CHUNK
NMCHUNK = BPW // MCHUNK

_mesh = plsc.VectorSubcoreMesh(core_axis_name="c", subcore_axis_name="s")


@functools.partial(
    pl.kernel,
    mesh=_mesh,
    out_type=jax.ShapeDtypeStruct((BATCH, D), jnp.float32),
    scratch_types=[
        pltpu.VMEM((NMCHUNK, MCHUNK), jnp.int32),
        pltpu.VMEM((NMCHUNK, MCHUNK, D), jnp.float32),
        pltpu.SemaphoreType.DMA,
    ],
    compiler_params=pltpu.CompilerParams(
        needs_layout_passes=False, use_tc_tiling_on_sc=False),
)
def _sc_movie(mid_hbm, mtab_hbm, rows_hbm, midx_v, rows_v, sem):
    wid = lax.axis_index("s") * NC + lax.axis_index("c")
    base = wid * BPW
    for j in range(NMCHUNK):
        pltpu.sync_copy(mid_hbm.at[pl.ds(base + j * MCHUNK, MCHUNK)],
                        midx_v.at[j])
    handles = []
    for j in range(NMCHUNK):
        handles.append(
            pltpu.async_copy(mtab_hbm.at[midx_v.at[j]], rows_v.at[j], sem))
    for j in range(NMCHUNK):
        handles[j].wait()
        pltpu.sync_copy(rows_v.at[j],
                        rows_hbm.at[pl.ds(base + j * MCHUNK, MCHUNK), :])


@functools.partial(
    pl.kernel,
    mesh=_mesh,
    out_type=jax.ShapeDtypeStruct((BATCH,), jnp.float32),
    scratch_types=[
        pltpu.VMEM((BPW,), jnp.int32),
        pltpu.VMEM((BPW * D,), jnp.float32),
        pltpu.VMEM((NBAND, NSUB, CHUNK * LANES), jnp.float32),
        pltpu.VMEM((NBAND, NSUB, CHUNK * LANES), jnp.float32),
        pltpu.VMEM((3 * LANES,), jnp.float32),
        pltpu.VMEM((BPW,), jnp.float32),
        pltpu.SemaphoreType.DMA,
        pltpu.SemaphoreType.DMA,
    ],
    compiler_params=pltpu.CompilerParams(needs_layout_passes=False),
)
def _sc_user(uid_hbm, utab_hbm, mfl_hbm, wb_hbm, out_hbm,
             uidx_v, mloc_v, ua_v, ub_v, wb_v, out_v, usem, msem):
    wid = lax.axis_index("s") * NC + lax.axis_index("c")
    base = wid * BPW

    pltpu.sync_copy(wb_hbm, wb_v)
    pltpu.sync_copy(uid_hbm.at[pl.ds(base, BPW)], uidx_v)
    mh = pltpu.async_copy(mfl_hbm.at[pl.ds(base * D, BPW * D)], mloc_v, msem)

    wv0 = wb_v[pl.ds(0, LANES)]
    wv1 = wb_v[pl.ds(LANES, LANES)]
    bias_vec = wb_v[pl.ds(2 * LANES, LANES)]
    lane_base = lax.iota(jnp.int32, LANES) * LANES
    lane_base_d = lax.iota(jnp.int32, LANES) * D

    def fire(c, urows_v):
        coff = pl.multiple_of(c * CHUNK, LANES)
        for g in range(CHUNK // LANES):
            uvec = uidx_v[pl.ds(coff + g * LANES, LANES)]
            for l in range(LANES):
                i = g * LANES + l
                pltpu.async_copy(
                    utab_hbm.at[:, :, pl.ds((uvec[l] // LANES) * LANES,
                                            LANES)],
                    urows_v.at[:, :, pl.ds(i * LANES, LANES)], usem)

    def drain_compute(c, urows_v):
        pltpu.make_async_copy(
            utab_hbm.at[:, :, pl.ds(0, CHUNK * LANES)], urows_v, usem).wait()
        coff = pl.multiple_of(c * CHUNK, LANES)
        for g in range(CHUNK // LANES):
            uvec = uidx_v[pl.ds(coff + g * LANES, LANES)]
            uoffs = lane_base + g * (LANES * LANES) + (uvec & (LANES - 1))
            moff0 = (coff + g * LANES) * D
            acc = bias_vec
            for j in range(NBAND):
                for s in range(NSUB):
                    d = j * NSUB + s
                    jv = jnp.full((LANES,), j, jnp.int32)
                    sv = jnp.full((LANES,), s, jnp.int32)
                    uv = plsc.load_gather(urows_v, [jv, sv, uoffs])
                    mv = plsc.load_gather(mloc_v, [lane_base_d + (moff0 + d)])
                    wsrc = wv0 if d < LANES else wv1
                    wd = jnp.broadcast_to(wsrc[d % LANES], (LANES,))
                    acc = acc + uv * mv * wd
            out_v[pl.ds(coff + g * LANES, LANES)] = acc

    fire(0, ua_v)
    mh.wait()

    def pair(c2, _):
        c = c2 * 2
        fire(c + 1, ub_v)
        drain_compute(c, ua_v)
        @pl.when(c + 2 < NCHUNK)
        def _():
            fire(c + 2, ua_v)
        drain_compute(c + 1, ub_v)
        return 0

    lax.fori_loop(0, NCHUNK // 2, pair, 0)

    pltpu.sync_copy(out_v, out_hbm.at[pl.ds(base, BPW)])


def kernel(user_id, movie_id, user_table, movie_table, dense_W, dense_b):
    n_users = user_table.shape[0]
    u3 = user_table.reshape(n_users, NBAND, NSUB).transpose(1, 2, 0)
    wb = jnp.concatenate([
        dense_W.reshape(-1),
        jnp.broadcast_to(dense_b.reshape(-1), (LANES,)),
    ])
    mrows = _sc_movie(movie_id, movie_table)
    out = _sc_user(user_id, u3, mrows.reshape(-1), wb)
    return out.reshape(BATCH, 1)
